# baseline (speedup 1.0000x reference)
import jax
import jax.numpy as jnp
from jax.experimental import pallas as pl

N = 10000
E = 320000
NUM_GRAPHS = 64
RATIO = 0.8


def _sage(x, ei, Wl, bl, Wr, edge_mask):
    src, dst = ei[0], ei[1]
    n = x.shape[0]
    w = edge_mask.astype(x.dtype)
    msum = jax.ops.segment_sum(x[src] * w[:, None], dst, num_segments=n)
    cnt = jax.ops.segment_sum(w, dst, num_segments=n)
    mean = msum / jnp.maximum(cnt, 1.0)[:, None]
    return mean @ Wl.T + bl + x @ Wr.T


def _topk_pool(x, ei, batch, p, node_mask, edge_mask):
    s = jnp.tanh((x @ p) / (jnp.linalg.norm(p) + 1e-16))
    n = x.shape[0]
    bkey = jnp.where(node_mask, batch, NUM_GRAPHS)
    counts = jax.ops.segment_sum(node_mask.astype(jnp.int32), batch, num_segments=NUM_GRAPHS)
    order = jnp.lexsort((-s, bkey))
    starts = jnp.concatenate([jnp.zeros((1,), counts.dtype), jnp.cumsum(counts)])
    rank_sorted = jnp.arange(n) - starts[bkey[order]]
    rank = jnp.zeros((n,), rank_sorted.dtype).at[order].set(rank_sorted)
    k_per = jnp.ceil(RATIO * counts).astype(rank.dtype)
    keep = node_mask & (rank < k_per[batch])
    new_x = x * s[:, None]
    new_edge_mask = edge_mask & keep[ei[0]] & keep[ei[1]]
    return new_x, keep, new_edge_mask


def _readout(x, batch, node_mask):
    bkey = jnp.where(node_mask, batch, NUM_GRAPHS)
    gm = jax.ops.segment_max(x, bkey, num_segments=NUM_GRAPHS)
    ssum = jax.ops.segment_sum(x, bkey, num_segments=NUM_GRAPHS)
    cnt = jax.ops.segment_sum(jnp.ones((x.shape[0],), jnp.float32), bkey, num_segments=NUM_GRAPHS)
    ga = ssum / jnp.maximum(cnt, 1.0)[:, None]
    return jnp.concatenate([gm, ga], axis=1)


def _mlp_kernel(r_ref, w1_ref, b1_ref, w2_ref, b2_ref, o_ref):
    h = jnp.maximum(r_ref[...] @ w1_ref[...].T + b1_ref[...], 0.0)
    o_ref[...] = h @ w2_ref[...].T + b2_ref[...]


def kernel(x, edge_index, batch, Wl1, bl1, Wr1, p1, Wl2, bl2, Wr2, p2, Wl3, bl3, Wr3, p3, Wl4, bl4, Wr4, p4, lw1, lb1, lw2, lb2):
    params = [(Wl1, bl1, Wr1, p1), (Wl2, bl2, Wr2, p2), (Wl3, bl3, Wr3, p3), (Wl4, bl4, Wr4, p4)]
    cur, ei, b = x, edge_index, batch
    node_mask = jnp.ones((x.shape[0],), jnp.bool_)
    edge_mask = jnp.ones((edge_index.shape[1],), jnp.bool_)
    r = None
    for (Wl, bl, Wr, p) in params:
        cur = jax.nn.relu(_sage(cur, ei, Wl, bl, Wr, edge_mask))
        cur, node_mask, edge_mask = _topk_pool(cur, ei, b, p, node_mask, edge_mask)
        ro = _readout(cur, b, node_mask)
        r = ro if r is None else r + ro
    out = pl.pallas_call(
        _mlp_kernel,
        out_shape=jax.ShapeDtypeStruct((NUM_GRAPHS, lw2.shape[0]), jnp.float32),
    )(r, lw1, lb1, lw2, lb2)
    return out


# speedup vs baseline: 8.3118x; 8.3118x over previous
import functools

import jax
import jax.numpy as jnp
from jax import lax
from jax.experimental import pallas as pl
from jax.experimental.pallas import tpu as pltpu
from jax.experimental.pallas import tpu_sc as plsc

N = 10000
E = 320000
DIM = 64
NUM_GRAPHS = 64
RATIO = 0.8

NC = 2
NS = 16
NW = NC * NS
CHUNK = 128
NCHUNK = 79
EPT = NCHUNK * CHUNK
EPAD = EPT * NW
NPAD = 10240
RPT = NPAD // NS
RPW = NPAD // NW
TRASH = N


def _mesh():
    return plsc.VectorSubcoreMesh(
        core_axis_name="c", subcore_axis_name="s", num_cores=NC, num_subcores=NS
    )


@functools.partial(
    pl.kernel,
    out_type=(
        jax.ShapeDtypeStruct((NC, NPAD, DIM), jnp.float32),
        jax.ShapeDtypeStruct((NC, NPAD, 16), jnp.float32),
    ),
    mesh=_mesh(),
    scratch_types=[
        pltpu.VMEM((NCHUNK, CHUNK), jnp.int32),
        pltpu.VMEM((NCHUNK, CHUNK), jnp.int32),
        pltpu.VMEM((NCHUNK, CHUNK), jnp.int32),
        pltpu.VMEM((NPAD,), jnp.int32),
        pltpu.VMEM((CHUNK, DIM), jnp.float32),
        pltpu.VMEM((CHUNK, 16), jnp.float32),
        pltpu.VMEM((64, DIM), jnp.float32),
        pltpu.VMEM((64, 16), jnp.float32),
        pltpu.VMEM_SHARED((NPAD, DIM), jnp.float32),
        pltpu.VMEM_SHARED((NPAD, 16), jnp.float32),
    ],
    compiler_params=pltpu.CompilerParams(
        needs_layout_passes=False, use_tc_tiling_on_sc=False),
)
def _sc_scatter(y_hbm, src_hbm, dst_hbm, keep_hbm, acc_out, cnt_out,
                src_v, dst_v, dstp_v, keep_v, rows_v, ones_v, za_v, zc_v,
                acc_sh, cnt_sh):
    cid = lax.axis_index("c")
    sid = lax.axis_index("s")
    wid = sid * NC + cid

    pltpu.sync_copy(src_hbm.at[wid], src_v)
    pltpu.sync_copy(dst_hbm.at[wid], dst_v)
    pltpu.sync_copy(keep_hbm, keep_v)

    zero16 = jnp.zeros((16,), jnp.float32)
    one16 = jnp.full((16,), 1.0, jnp.float32)

    def _fill(i, carry):
        for j in range(4):
            za_v[i, pl.ds(16 * j, 16)] = zero16
        zc_v[i] = zero16
        ones_v[i] = one16
        ones_v[64 + i] = one16
        return carry

    lax.fori_loop(0, 64, _fill, 0)

    base = sid * RPT
    for k in range(RPT // 64):
        pltpu.sync_copy(za_v, acc_sh.at[pl.ds(base + 64 * k, 64)])
        pltpu.sync_copy(zc_v, cnt_sh.at[pl.ds(base + 64 * k, 64)])

    def _mask_edges(j, carry):
        for i in range(8):
            sl = pl.ds(16 * i, 16)
            s_idx = src_v[j, sl]
            d_idx = dst_v[j, sl]
            ks = plsc.load_gather(keep_v, [s_idx])
            kd = plsc.load_gather(keep_v, [d_idx])
            ok = (ks * kd) > 0
            dstp_v[j, sl] = jnp.where(ok, d_idx, jnp.int32(TRASH))
        return carry

    lax.fori_loop(0, NCHUNK, _mask_edges, 0)
    plsc.subcore_barrier()

    def _edges(j, carry):
        pltpu.sync_copy(y_hbm.at[src_v.at[j]], rows_v)
        pltpu.sync_copy(rows_v, acc_sh.at[dstp_v.at[j]], add=True)
        pltpu.sync_copy(ones_v, cnt_sh.at[dstp_v.at[j]], add=True)
        return carry

    lax.fori_loop(0, NCHUNK, _edges, 0)
    plsc.subcore_barrier()

    pltpu.sync_copy(acc_sh.at[pl.ds(base, RPT)],
                    acc_out.at[cid, pl.ds(base, RPT)])
    pltpu.sync_copy(cnt_sh.at[pl.ds(base, RPT)],
                    cnt_out.at[cid, pl.ds(base, RPT)])


@functools.partial(
    pl.kernel,
    out_type=jax.ShapeDtypeStruct((NPAD, DIM), jnp.float32),
    mesh=_mesh(),
    scratch_types=[
        pltpu.VMEM((RPW, DIM), jnp.float32),
        pltpu.VMEM((RPW, DIM), jnp.float32),
        pltpu.VMEM((RPW, 16), jnp.float32),
        pltpu.VMEM((RPW, 16), jnp.float32),
        pltpu.VMEM((RPW, DIM), jnp.float32),
        pltpu.VMEM((RPW, DIM), jnp.float32),
    ],
    compiler_params=pltpu.CompilerParams(
        needs_layout_passes=False, use_tc_tiling_on_sc=False),
)
def _sc_finalize(acc_hbm, cnt_hbm, xr_hbm, h_out, a0, a1, c0, c1, xr_v, h_v):
    cid = lax.axis_index("c")
    sid = lax.axis_index("s")
    wid = sid * NC + cid
    base = wid * RPW
    pltpu.sync_copy(acc_hbm.at[0, pl.ds(base, RPW)], a0)
    pltpu.sync_copy(acc_hbm.at[1, pl.ds(base, RPW)], a1)
    pltpu.sync_copy(cnt_hbm.at[0, pl.ds(base, RPW)], c0)
    pltpu.sync_copy(cnt_hbm.at[1, pl.ds(base, RPW)], c1)
    pltpu.sync_copy(xr_hbm.at[pl.ds(base, RPW)], xr_v)

    def _rows(i, carry):
        cnt = c0[i] + c1[i]
        inv = 1.0 / jnp.maximum(cnt, 1.0)
        for j in range(4):
            sl = pl.ds(16 * j, 16)
            v = (a0[i, sl] + a1[i, sl]) * inv + xr_v[i, sl]
            h_v[i, sl] = jnp.maximum(v, 0.0)
        return carry

    lax.fori_loop(0, RPW, _rows, 0)
    pltpu.sync_copy(h_v, h_out.at[pl.ds(base, RPW)])


def _mm_body(x_ref, s_ref, wl_ref, bl_ref, wr_ref, y_ref, xr_ref):
    x = x_ref[...] * s_ref[...]
    y_ref[...] = jax.lax.dot_general(
        x, wl_ref[...], (((1,), (1,)), ((), ())),
        preferred_element_type=jnp.float32)
    xr_ref[...] = jax.lax.dot_general(
        x, wr_ref[...], (((1,), (1,)), ((), ())),
        preferred_element_type=jnp.float32) + bl_ref[...]


def _mm(x_pad, s_pad, Wl, bl, Wr):
    k = x_pad.shape[1]
    blk = NPAD // 4
    return pl.pallas_call(
        _mm_body,
        grid=(4,),
        in_specs=[
            pl.BlockSpec((blk, k), lambda i: (i, 0)),
            pl.BlockSpec((blk, 1), lambda i: (i, 0)),
            pl.BlockSpec((DIM, k), lambda i: (0, 0)),
            pl.BlockSpec((1, DIM), lambda i: (0, 0)),
            pl.BlockSpec((DIM, k), lambda i: (0, 0)),
        ],
        out_specs=[
            pl.BlockSpec((blk, DIM), lambda i: (i, 0)),
            pl.BlockSpec((blk, DIM), lambda i: (i, 0)),
        ],
        out_shape=[
            jax.ShapeDtypeStruct((NPAD, DIM), jnp.float32),
            jax.ShapeDtypeStruct((NPAD, DIM), jnp.float32),
        ],
    )(x_pad, s_pad[:, None], Wl, bl[None, :], Wr)


def _topk_keep(h, batch, p, node_mask):
    s = jnp.tanh((h @ p) / (jnp.linalg.norm(p) + 1e-16))
    n = h.shape[0]
    bkey = jnp.where(node_mask, batch, NUM_GRAPHS)
    counts = jax.ops.segment_sum(
        node_mask.astype(jnp.int32), batch, num_segments=NUM_GRAPHS)
    order = jnp.lexsort((-s, bkey))
    starts = jnp.concatenate([jnp.zeros((1,), counts.dtype), jnp.cumsum(counts)])
    rank_sorted = jnp.arange(n) - starts[bkey[order]]
    rank = jnp.zeros((n,), rank_sorted.dtype).at[order].set(rank_sorted)
    k_per = jnp.ceil(RATIO * counts).astype(rank.dtype)
    keep = node_mask & (rank < k_per[batch])
    return s, keep


def _readout(x, batch, node_mask):
    bkey = jnp.where(node_mask, batch, NUM_GRAPHS)
    gm = jax.ops.segment_max(x, bkey, num_segments=NUM_GRAPHS)
    ssum = jax.ops.segment_sum(x, bkey, num_segments=NUM_GRAPHS)
    cnt = jax.ops.segment_sum(
        jnp.ones((x.shape[0],), jnp.float32), bkey, num_segments=NUM_GRAPHS)
    ga = ssum / jnp.maximum(cnt, 1.0)[:, None]
    return jnp.concatenate([gm, ga], axis=1)


def _mlp_kernel(r_ref, w1_ref, b1_ref, w2_ref, b2_ref, o_ref):
    h = jnp.maximum(r_ref[...] @ w1_ref[...].T + b1_ref[...], 0.0)
    o_ref[...] = h @ w2_ref[...].T + b2_ref[...]


def kernel(x, edge_index, batch, Wl1, bl1, Wr1, p1, Wl2, bl2, Wr2, p2,
           Wl3, bl3, Wr3, p3, Wl4, bl4, Wr4, p4, lw1, lb1, lw2, lb2):
    params = [(Wl1, bl1, Wr1, p1), (Wl2, bl2, Wr2, p2),
              (Wl3, bl3, Wr3, p3), (Wl4, bl4, Wr4, p4)]
    src = jnp.concatenate(
        [edge_index[0], jnp.zeros((EPAD - E,), jnp.int32)]).reshape(NW, NCHUNK, CHUNK)
    dst = jnp.concatenate(
        [edge_index[1], jnp.full((EPAD - E,), TRASH, jnp.int32)]).reshape(NW, NCHUNK, CHUNK)

    keep = jnp.ones((N,), jnp.bool_)
    cur_pad = jnp.concatenate([x, jnp.zeros((NPAD - N, x.shape[1]), jnp.float32)])
    s_pad = jnp.ones((NPAD,), jnp.float32)
    r = None
    for (Wl, bl, Wr, p) in params:
        y_pad, xr_pad = _mm(cur_pad, s_pad, Wl, bl, Wr)
        keep_pad = jnp.concatenate(
            [keep.astype(jnp.int32), jnp.zeros((NPAD - N,), jnp.int32)])
        acc, cnt = _sc_scatter(y_pad, src, dst, keep_pad)
        h_pad = _sc_finalize(acc, cnt, xr_pad)
        h = h_pad[:N]
        s, keep = _topk_keep(h, batch, p, keep)
        cur = h * s[:, None]
        ro = _readout(cur, batch, keep)
        r = ro if r is None else r + ro
        cur_pad = jnp.concatenate(
            [h, jnp.zeros((NPAD - N, DIM), jnp.float32)])
        s_pad = jnp.concatenate([s, jnp.zeros((NPAD - N,), jnp.float32)])

    out = pl.pallas_call(
        _mlp_kernel,
        out_shape=jax.ShapeDtypeStruct((NUM_GRAPHS, lw2.shape[0]), jnp.float32),
    )(r, lw1, lb1, lw2, lb2)
    return out


# speedup vs baseline: 9.2008x; 1.1070x over previous
import functools

import jax
import jax.numpy as jnp
from jax import lax
from jax.experimental import pallas as pl
from jax.experimental.pallas import tpu as pltpu
from jax.experimental.pallas import tpu_sc as plsc

N = 10000
E = 320000
DIM = 64
NUM_GRAPHS = 64
RATIO = 0.8

NC = 2
NS = 16
NW = NC * NS
CHUNK = 128
NCHUNK = 79
EPT = NCHUNK * CHUNK
EPAD = EPT * NW
NPAD = 10240
RPT = NPAD // NS
RPW = NPAD // NW
TRASH = N


def _mesh():
    return plsc.VectorSubcoreMesh(
        core_axis_name="c", subcore_axis_name="s", num_cores=NC, num_subcores=NS
    )


@functools.partial(
    pl.kernel,
    out_type=(
        jax.ShapeDtypeStruct((NC, NPAD, DIM), jnp.float32),
        jax.ShapeDtypeStruct((NC, NPAD, 16), jnp.float32),
    ),
    mesh=_mesh(),
    scratch_types=[
        pltpu.VMEM((NCHUNK, CHUNK), jnp.int32),
        pltpu.VMEM((NCHUNK, CHUNK), jnp.int32),
        pltpu.VMEM((NCHUNK, CHUNK), jnp.int32),
        pltpu.VMEM((NPAD,), jnp.int32),
        pltpu.VMEM((2, CHUNK, DIM), jnp.float32),
        pltpu.VMEM((CHUNK, 16), jnp.float32),
        pltpu.VMEM((64, DIM), jnp.float32),
        pltpu.VMEM((64, 16), jnp.float32),
        pltpu.VMEM_SHARED((NPAD, DIM), jnp.float32),
        pltpu.VMEM_SHARED((NPAD, 16), jnp.float32),
        pltpu.SemaphoreType.DMA,
    ],
    compiler_params=pltpu.CompilerParams(
        needs_layout_passes=False, use_tc_tiling_on_sc=False),
)
def _sc_scatter(y_hbm, src_hbm, dst_hbm, keep_hbm, acc_out, cnt_out,
                src_v, dst_v, dstp_v, keep_v, rows_v, ones_v, za_v, zc_v,
                acc_sh, cnt_sh, sem):
    cid = lax.axis_index("c")
    sid = lax.axis_index("s")
    wid = sid * NC + cid

    pltpu.sync_copy(src_hbm.at[wid], src_v)
    pltpu.sync_copy(dst_hbm.at[wid], dst_v)
    pltpu.sync_copy(keep_hbm, keep_v)

    zero16 = jnp.zeros((16,), jnp.float32)
    one16 = jnp.full((16,), 1.0, jnp.float32)

    def _fill(i, carry):
        for j in range(4):
            za_v[i, pl.ds(16 * j, 16)] = zero16
        zc_v[i] = zero16
        ones_v[i] = one16
        ones_v[64 + i] = one16
        return carry

    lax.fori_loop(0, 64, _fill, 0)

    base = sid * RPT
    for k in range(RPT // 64):
        pltpu.sync_copy(za_v, acc_sh.at[pl.ds(base + 64 * k, 64)])
        pltpu.sync_copy(zc_v, cnt_sh.at[pl.ds(base + 64 * k, 64)])

    lane = jnp.arange(16, dtype=jnp.int32)
    def _mask_edges(j, carry):
        for i in range(8):
            sl = pl.ds(16 * i, 16)
            s_idx = src_v[j, sl]
            d_idx = dst_v[j, sl]
            ks = plsc.load_gather(keep_v, [s_idx])
            kd = plsc.load_gather(keep_v, [d_idx])
            ok = (ks * kd) > 0
            trash = TRASH + lax.rem(16 * (8 * j + i) + lane, NPAD - TRASH)
            dstp_v[j, sl] = jnp.where(ok, d_idx, trash)
        return carry

    lax.fori_loop(0, NCHUNK, _mask_edges, 0)
    plsc.subcore_barrier()

    pltpu.async_copy(y_hbm.at[src_v.at[0]], rows_v.at[0], sem)

    def _edges(j, carry):
        b = lax.rem(j, 2)
        pltpu.make_async_copy(y_hbm.at[src_v.at[j]], rows_v.at[b], sem).wait()

        @pl.when(j < NCHUNK - 1)
        def _prefetch():
            pltpu.async_copy(
                y_hbm.at[src_v.at[j + 1]], rows_v.at[lax.rem(j + 1, 2)], sem)

        pltpu.sync_copy(rows_v.at[b], acc_sh.at[dstp_v.at[j]], add=True)
        pltpu.sync_copy(ones_v, cnt_sh.at[dstp_v.at[j]], add=True)
        return carry

    lax.fori_loop(0, NCHUNK, _edges, 0)
    plsc.subcore_barrier()

    pltpu.sync_copy(acc_sh.at[pl.ds(base, RPT)],
                    acc_out.at[cid, pl.ds(base, RPT)])
    pltpu.sync_copy(cnt_sh.at[pl.ds(base, RPT)],
                    cnt_out.at[cid, pl.ds(base, RPT)])


@functools.partial(
    pl.kernel,
    out_type=jax.ShapeDtypeStruct((NPAD, DIM), jnp.float32),
    mesh=_mesh(),
    scratch_types=[
        pltpu.VMEM((RPW, DIM), jnp.float32),
        pltpu.VMEM((RPW, DIM), jnp.float32),
        pltpu.VMEM((RPW, 16), jnp.float32),
        pltpu.VMEM((RPW, 16), jnp.float32),
        pltpu.VMEM((RPW, DIM), jnp.float32),
        pltpu.VMEM((RPW, DIM), jnp.float32),
    ],
    compiler_params=pltpu.CompilerParams(
        needs_layout_passes=False, use_tc_tiling_on_sc=False),
)
def _sc_finalize(acc_hbm, cnt_hbm, xr_hbm, h_out, a0, a1, c0, c1, xr_v, h_v):
    cid = lax.axis_index("c")
    sid = lax.axis_index("s")
    wid = sid * NC + cid
    base = wid * RPW
    pltpu.sync_copy(acc_hbm.at[0, pl.ds(base, RPW)], a0)
    pltpu.sync_copy(acc_hbm.at[1, pl.ds(base, RPW)], a1)
    pltpu.sync_copy(cnt_hbm.at[0, pl.ds(base, RPW)], c0)
    pltpu.sync_copy(cnt_hbm.at[1, pl.ds(base, RPW)], c1)
    pltpu.sync_copy(xr_hbm.at[pl.ds(base, RPW)], xr_v)

    def _rows(i, carry):
        cnt = c0[i] + c1[i]
        inv = 1.0 / jnp.maximum(cnt, 1.0)
        for j in range(4):
            sl = pl.ds(16 * j, 16)
            v = (a0[i, sl] + a1[i, sl]) * inv + xr_v[i, sl]
            h_v[i, sl] = jnp.maximum(v, 0.0)
        return carry

    lax.fori_loop(0, RPW, _rows, 0)
    pltpu.sync_copy(h_v, h_out.at[pl.ds(base, RPW)])


def _mm_body(x_ref, s_ref, wl_ref, bl_ref, wr_ref, y_ref, xr_ref):
    x = x_ref[...] * s_ref[...]
    y_ref[...] = jax.lax.dot_general(
        x, wl_ref[...], (((1,), (1,)), ((), ())),
        preferred_element_type=jnp.float32)
    xr_ref[...] = jax.lax.dot_general(
        x, wr_ref[...], (((1,), (1,)), ((), ())),
        preferred_element_type=jnp.float32) + bl_ref[...]


def _mm(x_pad, s_pad, Wl, bl, Wr):
    k = x_pad.shape[1]
    blk = NPAD // 4
    return pl.pallas_call(
        _mm_body,
        grid=(4,),
        in_specs=[
            pl.BlockSpec((blk, k), lambda i: (i, 0)),
            pl.BlockSpec((blk, 1), lambda i: (i, 0)),
            pl.BlockSpec((DIM, k), lambda i: (0, 0)),
            pl.BlockSpec((1, DIM), lambda i: (0, 0)),
            pl.BlockSpec((DIM, k), lambda i: (0, 0)),
        ],
        out_specs=[
            pl.BlockSpec((blk, DIM), lambda i: (i, 0)),
            pl.BlockSpec((blk, DIM), lambda i: (i, 0)),
        ],
        out_shape=[
            jax.ShapeDtypeStruct((NPAD, DIM), jnp.float32),
            jax.ShapeDtypeStruct((NPAD, DIM), jnp.float32),
        ],
    )(x_pad, s_pad[:, None], Wl, bl[None, :], Wr)


def _topk_keep(h, batch, p, node_mask):
    s = jnp.tanh((h @ p) / (jnp.linalg.norm(p) + 1e-16))
    n = h.shape[0]
    bkey = jnp.where(node_mask, batch, NUM_GRAPHS)
    counts = jax.ops.segment_sum(
        node_mask.astype(jnp.int32), batch, num_segments=NUM_GRAPHS)
    order = jnp.lexsort((-s, bkey))
    starts = jnp.concatenate([jnp.zeros((1,), counts.dtype), jnp.cumsum(counts)])
    rank_sorted = jnp.arange(n) - starts[bkey[order]]
    rank = jnp.zeros((n,), rank_sorted.dtype).at[order].set(rank_sorted)
    k_per = jnp.ceil(RATIO * counts).astype(rank.dtype)
    keep = node_mask & (rank < k_per[batch])
    return s, keep


def _readout(x, batch, node_mask):
    bkey = jnp.where(node_mask, batch, NUM_GRAPHS)
    gm = jax.ops.segment_max(x, bkey, num_segments=NUM_GRAPHS)
    ssum = jax.ops.segment_sum(x, bkey, num_segments=NUM_GRAPHS)
    cnt = jax.ops.segment_sum(
        jnp.ones((x.shape[0],), jnp.float32), bkey, num_segments=NUM_GRAPHS)
    ga = ssum / jnp.maximum(cnt, 1.0)[:, None]
    return jnp.concatenate([gm, ga], axis=1)


def _mlp_kernel(r_ref, w1_ref, b1_ref, w2_ref, b2_ref, o_ref):
    h = jnp.maximum(r_ref[...] @ w1_ref[...].T + b1_ref[...], 0.0)
    o_ref[...] = h @ w2_ref[...].T + b2_ref[...]


def kernel(x, edge_index, batch, Wl1, bl1, Wr1, p1, Wl2, bl2, Wr2, p2,
           Wl3, bl3, Wr3, p3, Wl4, bl4, Wr4, p4, lw1, lb1, lw2, lb2):
    params = [(Wl1, bl1, Wr1, p1), (Wl2, bl2, Wr2, p2),
              (Wl3, bl3, Wr3, p3), (Wl4, bl4, Wr4, p4)]
    src = jnp.concatenate(
        [edge_index[0], jnp.zeros((EPAD - E,), jnp.int32)]).reshape(NW, NCHUNK, CHUNK)
    dst = jnp.concatenate(
        [edge_index[1], jnp.full((EPAD - E,), TRASH, jnp.int32)]).reshape(NW, NCHUNK, CHUNK)

    keep = jnp.ones((N,), jnp.bool_)
    cur_pad = jnp.concatenate([x, jnp.zeros((NPAD - N, x.shape[1]), jnp.float32)])
    s_pad = jnp.ones((NPAD,), jnp.float32)
    r = None
    for (Wl, bl, Wr, p) in params:
        y_pad, xr_pad = _mm(cur_pad, s_pad, Wl, bl, Wr)
        keep_pad = jnp.concatenate(
            [keep.astype(jnp.int32), jnp.zeros((NPAD - N,), jnp.int32)])
        acc, cnt = _sc_scatter(y_pad, src, dst, keep_pad)
        h_pad = _sc_finalize(acc, cnt, xr_pad)
        h = h_pad[:N]
        s, keep = _topk_keep(h, batch, p, keep)
        cur = h * s[:, None]
        ro = _readout(cur, batch, keep)
        r = ro if r is None else r + ro
        cur_pad = jnp.concatenate(
            [h, jnp.zeros((NPAD - N, DIM), jnp.float32)])
        s_pad = jnp.concatenate([s, jnp.zeros((NPAD - N,), jnp.float32)])

    out = pl.pallas_call(
        _mlp_kernel,
        out_shape=jax.ShapeDtypeStruct((NUM_GRAPHS, lw2.shape[0]), jnp.float32),
    )(r, lw1, lb1, lw2, lb2)
    return out


# speedup vs baseline: 11.0448x; 1.2004x over previous
import functools

import jax
import jax.numpy as jnp
from jax import lax
from jax.experimental import pallas as pl
from jax.experimental.pallas import tpu as pltpu
from jax.experimental.pallas import tpu_sc as plsc

N = 10000
E = 320000
DIM = 64
NUM_GRAPHS = 64
RATIO = 0.8

NC = 2
NS = 16
NW = NC * NS
CHUNK = 128
NCHUNK = 79
EPT = NCHUNK * CHUNK
EPAD = EPT * NW
NPAD = 10240
RPT = NPAD // NS
RPW = NPAD // NW
TRASH = N


def _mesh():
    return plsc.VectorSubcoreMesh(
        core_axis_name="c", subcore_axis_name="s", num_cores=NC, num_subcores=NS
    )


@functools.partial(
    pl.kernel,
    out_type=(
        jax.ShapeDtypeStruct((NC, NPAD, DIM), jnp.float32),
        jax.ShapeDtypeStruct((NC, NPAD, 16), jnp.float32),
    ),
    mesh=_mesh(),
    scratch_types=[
        pltpu.VMEM((NCHUNK, CHUNK), jnp.int32),
        pltpu.VMEM((NCHUNK, CHUNK), jnp.int32),
        pltpu.VMEM((NCHUNK, CHUNK), jnp.int32),
        pltpu.VMEM((NPAD,), jnp.int32),
        pltpu.VMEM((2, CHUNK, DIM), jnp.float32),
        pltpu.VMEM((CHUNK, 16), jnp.float32),
        pltpu.VMEM((64, DIM), jnp.float32),
        pltpu.VMEM((64, 16), jnp.float32),
        pltpu.VMEM_SHARED((NPAD, DIM), jnp.float32),
        pltpu.VMEM_SHARED((NPAD, 16), jnp.float32),
        pltpu.SemaphoreType.DMA,
    ],
    compiler_params=pltpu.CompilerParams(
        needs_layout_passes=False, use_tc_tiling_on_sc=False),
)
def _sc_scatter(y_hbm, src_hbm, dst_hbm, keep_hbm, acc_out, cnt_out,
                src_v, dst_v, dstp_v, keep_v, rows_v, ones_v, za_v, zc_v,
                acc_sh, cnt_sh, sem):
    cid = lax.axis_index("c")
    sid = lax.axis_index("s")
    wid = sid * NC + cid

    pltpu.sync_copy(src_hbm.at[wid], src_v)
    pltpu.sync_copy(dst_hbm.at[wid], dst_v)
    pltpu.sync_copy(keep_hbm, keep_v)

    zero16 = jnp.zeros((16,), jnp.float32)
    one16 = jnp.full((16,), 1.0, jnp.float32)

    def _fill(i, carry):
        for j in range(4):
            za_v[i, pl.ds(16 * j, 16)] = zero16
        zc_v[i] = zero16
        ones_v[i] = one16
        ones_v[64 + i] = one16
        return carry

    lax.fori_loop(0, 64, _fill, 0)

    base = sid * RPT
    for k in range(RPT // 64):
        pltpu.sync_copy(za_v, acc_sh.at[pl.ds(base + 64 * k, 64)])
        pltpu.sync_copy(zc_v, cnt_sh.at[pl.ds(base + 64 * k, 64)])

    lane = jnp.arange(16, dtype=jnp.int32)
    def _mask_edges(j, carry):
        for i in range(8):
            sl = pl.ds(16 * i, 16)
            s_idx = src_v[j, sl]
            d_idx = dst_v[j, sl]
            ks = plsc.load_gather(keep_v, [s_idx])
            kd = plsc.load_gather(keep_v, [d_idx])
            ok = (ks * kd) > 0
            trash = TRASH + lax.rem(16 * (8 * j + i) + lane, NPAD - TRASH)
            dstp_v[j, sl] = jnp.where(ok, d_idx, trash)
        return carry

    lax.fori_loop(0, NCHUNK, _mask_edges, 0)
    plsc.subcore_barrier()

    pltpu.async_copy(y_hbm.at[src_v.at[0]], rows_v.at[0], sem)

    def _edges(j, carry):
        b = lax.rem(j, 2)
        pltpu.make_async_copy(y_hbm.at[src_v.at[j]], rows_v.at[b], sem).wait()

        @pl.when(j < NCHUNK - 1)
        def _prefetch():
            pltpu.async_copy(
                y_hbm.at[src_v.at[j + 1]], rows_v.at[lax.rem(j + 1, 2)], sem)

        pltpu.sync_copy(rows_v.at[b], acc_sh.at[dstp_v.at[j]], add=True)
        pltpu.sync_copy(ones_v, cnt_sh.at[dstp_v.at[j]], add=True)
        return carry

    lax.fori_loop(0, NCHUNK, _edges, 0)
    plsc.subcore_barrier()

    pltpu.sync_copy(acc_sh.at[pl.ds(base, RPT)],
                    acc_out.at[cid, pl.ds(base, RPT)])
    pltpu.sync_copy(cnt_sh.at[pl.ds(base, RPT)],
                    cnt_out.at[cid, pl.ds(base, RPT)])


@functools.partial(
    pl.kernel,
    out_type=jax.ShapeDtypeStruct((NPAD, DIM), jnp.float32),
    mesh=_mesh(),
    scratch_types=[
        pltpu.VMEM((RPW, DIM), jnp.float32),
        pltpu.VMEM((RPW, DIM), jnp.float32),
        pltpu.VMEM((RPW, 16), jnp.float32),
        pltpu.VMEM((RPW, 16), jnp.float32),
        pltpu.VMEM((RPW, DIM), jnp.float32),
        pltpu.VMEM((RPW, DIM), jnp.float32),
    ],
    compiler_params=pltpu.CompilerParams(
        needs_layout_passes=False, use_tc_tiling_on_sc=False),
)
def _sc_finalize(acc_hbm, cnt_hbm, xr_hbm, h_out, a0, a1, c0, c1, xr_v, h_v):
    cid = lax.axis_index("c")
    sid = lax.axis_index("s")
    wid = sid * NC + cid
    base = wid * RPW
    pltpu.sync_copy(acc_hbm.at[0, pl.ds(base, RPW)], a0)
    pltpu.sync_copy(acc_hbm.at[1, pl.ds(base, RPW)], a1)
    pltpu.sync_copy(cnt_hbm.at[0, pl.ds(base, RPW)], c0)
    pltpu.sync_copy(cnt_hbm.at[1, pl.ds(base, RPW)], c1)
    pltpu.sync_copy(xr_hbm.at[pl.ds(base, RPW)], xr_v)

    def _rows(i, carry):
        cnt = c0[i] + c1[i]
        inv = 1.0 / jnp.maximum(cnt, 1.0)
        for j in range(4):
            sl = pl.ds(16 * j, 16)
            v = (a0[i, sl] + a1[i, sl]) * inv + xr_v[i, sl]
            h_v[i, sl] = jnp.maximum(v, 0.0)
        return carry

    lax.fori_loop(0, RPW, _rows, 0)
    pltpu.sync_copy(h_v, h_out.at[pl.ds(base, RPW)])


def _mm_body(x_ref, s_ref, wl_ref, bl_ref, wr_ref, y_ref, xr_ref):
    x = x_ref[...] * s_ref[...]
    y_ref[...] = jax.lax.dot_general(
        x, wl_ref[...], (((1,), (1,)), ((), ())),
        preferred_element_type=jnp.float32)
    xr_ref[...] = jax.lax.dot_general(
        x, wr_ref[...], (((1,), (1,)), ((), ())),
        preferred_element_type=jnp.float32) + bl_ref[...]


def _mm(x_pad, s_pad, Wl, bl, Wr):
    k = x_pad.shape[1]
    blk = NPAD // 4
    return pl.pallas_call(
        _mm_body,
        grid=(4,),
        in_specs=[
            pl.BlockSpec((blk, k), lambda i: (i, 0)),
            pl.BlockSpec((blk, 1), lambda i: (i, 0)),
            pl.BlockSpec((DIM, k), lambda i: (0, 0)),
            pl.BlockSpec((1, DIM), lambda i: (0, 0)),
            pl.BlockSpec((DIM, k), lambda i: (0, 0)),
        ],
        out_specs=[
            pl.BlockSpec((blk, DIM), lambda i: (i, 0)),
            pl.BlockSpec((blk, DIM), lambda i: (i, 0)),
        ],
        out_shape=[
            jax.ShapeDtypeStruct((NPAD, DIM), jnp.float32),
            jax.ShapeDtypeStruct((NPAD, DIM), jnp.float32),
        ],
    )(x_pad, s_pad[:, None], Wl, bl[None, :], Wr)


def _topk_keep(h, batch, p, node_mask, nstart):
    s = jnp.tanh((h @ p) / (jnp.linalg.norm(p) + 1e-16))
    bkey = jnp.where(node_mask, batch, NUM_GRAPHS)
    counts = jax.ops.segment_sum(
        node_mask.astype(jnp.int32), batch, num_segments=NUM_GRAPHS)
    starts = jnp.concatenate([jnp.zeros((1,), counts.dtype), jnp.cumsum(counts)])
    _, negs_sorted = jax.lax.sort((bkey, -s), num_keys=2)
    k_per = jnp.ceil(RATIO * counts).astype(jnp.int32)
    pos = starts[:NUM_GRAPHS] + jnp.maximum(k_per, 1) - 1
    thr = -negs_sorted[pos]
    thr_b = thr[batch]
    sgt = node_mask & (s > thr_b)
    tie = node_mask & (s == thr_b)
    c_gt = jax.ops.segment_sum(
        sgt.astype(jnp.int32), batch, num_segments=NUM_GRAPHS)
    tcum_ex = jnp.cumsum(tie.astype(jnp.int32)) - tie.astype(jnp.int32)
    base = tcum_ex[nstart[:NUM_GRAPHS]]
    rank_tie = tcum_ex - base[batch]
    keep = sgt | (tie & (rank_tie < (k_per - c_gt)[batch]))
    return s, keep


def _readout(x, batch, node_mask):
    bkey = jnp.where(node_mask, batch, NUM_GRAPHS)
    gm = jax.ops.segment_max(x, bkey, num_segments=NUM_GRAPHS)
    ssum = jax.ops.segment_sum(x, bkey, num_segments=NUM_GRAPHS)
    cnt = jax.ops.segment_sum(
        jnp.ones((x.shape[0],), jnp.float32), bkey, num_segments=NUM_GRAPHS)
    ga = ssum / jnp.maximum(cnt, 1.0)[:, None]
    return jnp.concatenate([gm, ga], axis=1)


def _mlp_kernel(r_ref, w1_ref, b1_ref, w2_ref, b2_ref, o_ref):
    h = jnp.maximum(r_ref[...] @ w1_ref[...].T + b1_ref[...], 0.0)
    o_ref[...] = h @ w2_ref[...].T + b2_ref[...]


def kernel(x, edge_index, batch, Wl1, bl1, Wr1, p1, Wl2, bl2, Wr2, p2,
           Wl3, bl3, Wr3, p3, Wl4, bl4, Wr4, p4, lw1, lb1, lw2, lb2):
    params = [(Wl1, bl1, Wr1, p1), (Wl2, bl2, Wr2, p2),
              (Wl3, bl3, Wr3, p3), (Wl4, bl4, Wr4, p4)]
    src = jnp.concatenate(
        [edge_index[0], jnp.zeros((EPAD - E,), jnp.int32)]).reshape(NW, NCHUNK, CHUNK)
    dst = jnp.concatenate(
        [edge_index[1], jnp.full((EPAD - E,), TRASH, jnp.int32)]).reshape(NW, NCHUNK, CHUNK)

    keep = jnp.ones((N,), jnp.bool_)
    nodes_per_graph = jax.ops.segment_sum(
        jnp.ones((N,), jnp.int32), batch, num_segments=NUM_GRAPHS)
    nstart = jnp.concatenate(
        [jnp.zeros((1,), jnp.int32), jnp.cumsum(nodes_per_graph)])
    cur_pad = jnp.concatenate([x, jnp.zeros((NPAD - N, x.shape[1]), jnp.float32)])
    s_pad = jnp.ones((NPAD,), jnp.float32)
    r = None
    for (Wl, bl, Wr, p) in params:
        y_pad, xr_pad = _mm(cur_pad, s_pad, Wl, bl, Wr)
        keep_pad = jnp.concatenate(
            [keep.astype(jnp.int32), jnp.zeros((NPAD - N,), jnp.int32)])
        acc, cnt = _sc_scatter(y_pad, src, dst, keep_pad)
        h_pad = _sc_finalize(acc, cnt, xr_pad)
        h = h_pad[:N]
        s, keep = _topk_keep(h, batch, p, keep, nstart)
        cur = h * s[:, None]
        ro = _readout(cur, batch, keep)
        r = ro if r is None else r + ro
        cur_pad = jnp.concatenate(
            [h, jnp.zeros((NPAD - N, DIM), jnp.float32)])
        s_pad = jnp.concatenate([s, jnp.zeros((NPAD - N,), jnp.float32)])

    out = pl.pallas_call(
        _mlp_kernel,
        out_shape=jax.ShapeDtypeStruct((NUM_GRAPHS, lw2.shape[0]), jnp.float32),
    )(r, lw1, lb1, lw2, lb2)
    return out


# speedup vs baseline: 11.0494x; 1.0004x over previous
import functools

import jax
import jax.numpy as jnp
from jax import lax
from jax.experimental import pallas as pl
from jax.experimental.pallas import tpu as pltpu
from jax.experimental.pallas import tpu_sc as plsc

N = 10000
E = 320000
DIM = 64
NUM_GRAPHS = 64
RATIO = 0.8

NC = 2
NS = 16
NW = NC * NS
CHUNK = 128
NCHUNK = 79
EPT = NCHUNK * CHUNK
EPAD = EPT * NW
NPAD = 10240
RPT = NPAD // NS
RPW = NPAD // NW
TRASH = N


def _mesh():
    return plsc.VectorSubcoreMesh(
        core_axis_name="c", subcore_axis_name="s", num_cores=NC, num_subcores=NS
    )


@functools.partial(
    pl.kernel,
    out_type=(
        jax.ShapeDtypeStruct((NC, NPAD, DIM), jnp.float32),
        jax.ShapeDtypeStruct((NC, NPAD, 16), jnp.float32),
    ),
    mesh=_mesh(),
    scratch_types=[
        pltpu.VMEM((NCHUNK, CHUNK), jnp.int32),
        pltpu.VMEM((NCHUNK, CHUNK), jnp.int32),
        pltpu.VMEM((NCHUNK + 1, CHUNK), jnp.int32),
        pltpu.VMEM((NCHUNK + 1, CHUNK), jnp.int32),
        pltpu.VMEM((NPAD,), jnp.int32),
        pltpu.VMEM((2, CHUNK, DIM), jnp.float32),
        pltpu.VMEM((CHUNK, 16), jnp.float32),
        pltpu.VMEM((64, DIM), jnp.float32),
        pltpu.VMEM((64, 16), jnp.float32),
        pltpu.VMEM_SHARED((NPAD, DIM), jnp.float32),
        pltpu.VMEM_SHARED((NPAD, 16), jnp.float32),
        pltpu.SemaphoreType.DMA,
    ],
    compiler_params=pltpu.CompilerParams(
        needs_layout_passes=False, use_tc_tiling_on_sc=False),
)
def _sc_scatter(y_hbm, src_hbm, dst_hbm, keep_hbm, acc_out, cnt_out,
                src_v, dst_v, csrc_v, cdst_v, keep_v, rows_v, ones_v, za_v,
                zc_v, acc_sh, cnt_sh, sem):
    cid = lax.axis_index("c")
    sid = lax.axis_index("s")
    wid = sid * NC + cid

    pltpu.sync_copy(src_hbm.at[wid], src_v)
    pltpu.sync_copy(dst_hbm.at[wid], dst_v)
    pltpu.sync_copy(keep_hbm, keep_v)

    zero16 = jnp.zeros((16,), jnp.float32)
    one16 = jnp.full((16,), 1.0, jnp.float32)

    def _fill(i, carry):
        for j in range(4):
            za_v[i, pl.ds(16 * j, 16)] = zero16
        zc_v[i] = zero16
        ones_v[i] = one16
        ones_v[64 + i] = one16
        return carry

    lax.fori_loop(0, 64, _fill, 0)

    base = sid * RPT
    for k in range(RPT // 64):
        pltpu.sync_copy(za_v, acc_sh.at[pl.ds(base + 64 * k, 64)])
        pltpu.sync_copy(zc_v, cnt_sh.at[pl.ds(base + 64 * k, 64)])

    lane = jnp.arange(16, dtype=jnp.int32)

    def _compact(j, off):
        for i in range(8):
            sl = pl.ds(16 * i, 16)
            s_idx = src_v[j, sl]
            d_idx = dst_v[j, sl]
            ks = plsc.load_gather(keep_v, [s_idx])
            kd = plsc.load_gather(keep_v, [d_idx])
            ok = (ks * kd) > 0
            cs = plsc.cumsum(ok.astype(jnp.int32))
            pos = off + cs - 1
            row = lax.shift_right_logical(pos, 7)
            col = lax.bitwise_and(pos, 127)
            plsc.store_scatter(csrc_v, [row, col], s_idx, mask=ok)
            plsc.store_scatter(cdst_v, [row, col], d_idx, mask=ok)
            off = off + jnp.max(cs)
        return off

    off = lax.fori_loop(0, NCHUNK, _compact, jnp.int32(0))

    for k in range(8):
        pos = off + 16 * k + lane
        row = lax.shift_right_logical(pos, 7)
        col = lax.bitwise_and(pos, 127)
        plsc.store_scatter(csrc_v, [row, col], jnp.zeros((16,), jnp.int32))
        plsc.store_scatter(cdst_v, [row, col], TRASH + lane)
    nchunks = lax.shift_right_logical(off + 127, 7)
    plsc.subcore_barrier()

    @pl.when(nchunks > 0)
    def _prime():
        pltpu.async_copy(y_hbm.at[csrc_v.at[0]], rows_v.at[0], sem)

    def _edges(j, carry):
        b = lax.rem(j, 2)
        pltpu.make_async_copy(y_hbm.at[csrc_v.at[j]], rows_v.at[b], sem).wait()

        @pl.when(j < nchunks - 1)
        def _prefetch():
            pltpu.async_copy(
                y_hbm.at[csrc_v.at[j + 1]], rows_v.at[lax.rem(j + 1, 2)], sem)

        pltpu.sync_copy(rows_v.at[b], acc_sh.at[cdst_v.at[j]], add=True)
        pltpu.sync_copy(ones_v, cnt_sh.at[cdst_v.at[j]], add=True)
        return carry

    lax.fori_loop(0, nchunks, _edges, 0)
    plsc.subcore_barrier()

    pltpu.sync_copy(acc_sh.at[pl.ds(base, RPT)],
                    acc_out.at[cid, pl.ds(base, RPT)])
    pltpu.sync_copy(cnt_sh.at[pl.ds(base, RPT)],
                    cnt_out.at[cid, pl.ds(base, RPT)])


@functools.partial(
    pl.kernel,
    out_type=jax.ShapeDtypeStruct((NPAD, DIM), jnp.float32),
    mesh=_mesh(),
    scratch_types=[
        pltpu.VMEM((RPW, DIM), jnp.float32),
        pltpu.VMEM((RPW, DIM), jnp.float32),
        pltpu.VMEM((RPW, 16), jnp.float32),
        pltpu.VMEM((RPW, 16), jnp.float32),
        pltpu.VMEM((RPW, DIM), jnp.float32),
        pltpu.VMEM((RPW, DIM), jnp.float32),
    ],
    compiler_params=pltpu.CompilerParams(
        needs_layout_passes=False, use_tc_tiling_on_sc=False),
)
def _sc_finalize(acc_hbm, cnt_hbm, xr_hbm, h_out, a0, a1, c0, c1, xr_v, h_v):
    cid = lax.axis_index("c")
    sid = lax.axis_index("s")
    wid = sid * NC + cid
    base = wid * RPW
    pltpu.sync_copy(acc_hbm.at[0, pl.ds(base, RPW)], a0)
    pltpu.sync_copy(acc_hbm.at[1, pl.ds(base, RPW)], a1)
    pltpu.sync_copy(cnt_hbm.at[0, pl.ds(base, RPW)], c0)
    pltpu.sync_copy(cnt_hbm.at[1, pl.ds(base, RPW)], c1)
    pltpu.sync_copy(xr_hbm.at[pl.ds(base, RPW)], xr_v)

    def _rows(i, carry):
        cnt = c0[i] + c1[i]
        inv = 1.0 / jnp.maximum(cnt, 1.0)
        for j in range(4):
            sl = pl.ds(16 * j, 16)
            v = (a0[i, sl] + a1[i, sl]) * inv + xr_v[i, sl]
            h_v[i, sl] = jnp.maximum(v, 0.0)
        return carry

    lax.fori_loop(0, RPW, _rows, 0)
    pltpu.sync_copy(h_v, h_out.at[pl.ds(base, RPW)])


def _mm_body(x_ref, s_ref, wl_ref, bl_ref, wr_ref, y_ref, xr_ref):
    x = x_ref[...] * s_ref[...]
    y_ref[...] = jax.lax.dot_general(
        x, wl_ref[...], (((1,), (1,)), ((), ())),
        preferred_element_type=jnp.float32)
    xr_ref[...] = jax.lax.dot_general(
        x, wr_ref[...], (((1,), (1,)), ((), ())),
        preferred_element_type=jnp.float32) + bl_ref[...]


def _mm(x_pad, s_pad, Wl, bl, Wr):
    k = x_pad.shape[1]
    blk = NPAD // 4
    return pl.pallas_call(
        _mm_body,
        grid=(4,),
        in_specs=[
            pl.BlockSpec((blk, k), lambda i: (i, 0)),
            pl.BlockSpec((blk, 1), lambda i: (i, 0)),
            pl.BlockSpec((DIM, k), lambda i: (0, 0)),
            pl.BlockSpec((1, DIM), lambda i: (0, 0)),
            pl.BlockSpec((DIM, k), lambda i: (0, 0)),
        ],
        out_specs=[
            pl.BlockSpec((blk, DIM), lambda i: (i, 0)),
            pl.BlockSpec((blk, DIM), lambda i: (i, 0)),
        ],
        out_shape=[
            jax.ShapeDtypeStruct((NPAD, DIM), jnp.float32),
            jax.ShapeDtypeStruct((NPAD, DIM), jnp.float32),
        ],
    )(x_pad, s_pad[:, None], Wl, bl[None, :], Wr)


def _topk_keep(h, batch, p, node_mask, nstart):
    s = jnp.tanh((h @ p) / (jnp.linalg.norm(p) + 1e-16))
    bkey = jnp.where(node_mask, batch, NUM_GRAPHS)
    counts = jax.ops.segment_sum(
        node_mask.astype(jnp.int32), batch, num_segments=NUM_GRAPHS)
    starts = jnp.concatenate([jnp.zeros((1,), counts.dtype), jnp.cumsum(counts)])
    _, negs_sorted = jax.lax.sort((bkey, -s), num_keys=2)
    k_per = jnp.ceil(RATIO * counts).astype(jnp.int32)
    pos = starts[:NUM_GRAPHS] + jnp.maximum(k_per, 1) - 1
    thr = -negs_sorted[pos]
    thr_b = thr[batch]
    sgt = node_mask & (s > thr_b)
    tie = node_mask & (s == thr_b)
    c_gt = jax.ops.segment_sum(
        sgt.astype(jnp.int32), batch, num_segments=NUM_GRAPHS)
    tcum_ex = jnp.cumsum(tie.astype(jnp.int32)) - tie.astype(jnp.int32)
    base = tcum_ex[nstart[:NUM_GRAPHS]]
    rank_tie = tcum_ex - base[batch]
    keep = sgt | (tie & (rank_tie < (k_per - c_gt)[batch]))
    return s, keep


def _readout(x, batch, node_mask):
    bkey = jnp.where(node_mask, batch, NUM_GRAPHS)
    gm = jax.ops.segment_max(x, bkey, num_segments=NUM_GRAPHS)
    ssum = jax.ops.segment_sum(x, bkey, num_segments=NUM_GRAPHS)
    cnt = jax.ops.segment_sum(
        jnp.ones((x.shape[0],), jnp.float32), bkey, num_segments=NUM_GRAPHS)
    ga = ssum / jnp.maximum(cnt, 1.0)[:, None]
    return jnp.concatenate([gm, ga], axis=1)


def _mlp_kernel(r_ref, w1_ref, b1_ref, w2_ref, b2_ref, o_ref):
    h = jnp.maximum(r_ref[...] @ w1_ref[...].T + b1_ref[...], 0.0)
    o_ref[...] = h @ w2_ref[...].T + b2_ref[...]


def kernel(x, edge_index, batch, Wl1, bl1, Wr1, p1, Wl2, bl2, Wr2, p2,
           Wl3, bl3, Wr3, p3, Wl4, bl4, Wr4, p4, lw1, lb1, lw2, lb2):
    params = [(Wl1, bl1, Wr1, p1), (Wl2, bl2, Wr2, p2),
              (Wl3, bl3, Wr3, p3), (Wl4, bl4, Wr4, p4)]
    src = jnp.concatenate(
        [edge_index[0], jnp.zeros((EPAD - E,), jnp.int32)]).reshape(NW, NCHUNK, CHUNK)
    dst = jnp.concatenate(
        [edge_index[1], jnp.full((EPAD - E,), TRASH, jnp.int32)]).reshape(NW, NCHUNK, CHUNK)

    keep = jnp.ones((N,), jnp.bool_)
    nodes_per_graph = jax.ops.segment_sum(
        jnp.ones((N,), jnp.int32), batch, num_segments=NUM_GRAPHS)
    nstart = jnp.concatenate(
        [jnp.zeros((1,), jnp.int32), jnp.cumsum(nodes_per_graph)])
    cur_pad = jnp.concatenate([x, jnp.zeros((NPAD - N, x.shape[1]), jnp.float32)])
    s_pad = jnp.ones((NPAD,), jnp.float32)
    r = None
    for (Wl, bl, Wr, p) in params:
        y_pad, xr_pad = _mm(cur_pad, s_pad, Wl, bl, Wr)
        keep_pad = jnp.concatenate(
            [keep.astype(jnp.int32), jnp.zeros((NPAD - N,), jnp.int32)])
        acc, cnt = _sc_scatter(y_pad, src, dst, keep_pad)
        h_pad = _sc_finalize(acc, cnt, xr_pad)
        h = h_pad[:N]
        s, keep = _topk_keep(h, batch, p, keep, nstart)
        cur = h * s[:, None]
        ro = _readout(cur, batch, keep)
        r = ro if r is None else r + ro
        cur_pad = jnp.concatenate(
            [h, jnp.zeros((NPAD - N, DIM), jnp.float32)])
        s_pad = jnp.concatenate([s, jnp.zeros((NPAD - N,), jnp.float32)])

    out = pl.pallas_call(
        _mlp_kernel,
        out_shape=jax.ShapeDtypeStruct((NUM_GRAPHS, lw2.shape[0]), jnp.float32),
    )(r, lw1, lb1, lw2, lb2)
    return out


# speedup vs baseline: 14.9441x; 1.3525x over previous
import functools

import jax
import jax.numpy as jnp
from jax import lax
from jax.experimental import pallas as pl
from jax.experimental.pallas import tpu as pltpu
from jax.experimental.pallas import tpu_sc as plsc

N = 10000
E = 320000
DIM = 64
NUM_GRAPHS = 64
RATIO = 0.8

NC = 2
NS = 16
NW = NC * NS
CHUNK = 128
NCHUNK = 79
EPT = NCHUNK * CHUNK
EPAD = EPT * NW
NPAD = 10240
RPT = NPAD // NS
RPW = NPAD // NW
TRASH = N


def _mesh():
    return plsc.VectorSubcoreMesh(
        core_axis_name="c", subcore_axis_name="s", num_cores=NC, num_subcores=NS
    )


@functools.partial(
    pl.kernel,
    out_type=(
        jax.ShapeDtypeStruct((NC, NPAD, DIM), jnp.float32),
        jax.ShapeDtypeStruct((NC, NPAD, 16), jnp.float32),
    ),
    mesh=_mesh(),
    scratch_types=[
        pltpu.VMEM((NCHUNK, CHUNK), jnp.int32),
        pltpu.VMEM((NCHUNK, CHUNK), jnp.int32),
        pltpu.VMEM((NCHUNK + 1, CHUNK), jnp.int32),
        pltpu.VMEM((NCHUNK + 1, CHUNK), jnp.int32),
        pltpu.VMEM((NPAD,), jnp.int32),
        pltpu.VMEM((2, CHUNK, DIM), jnp.float32),
        pltpu.VMEM((CHUNK, 16), jnp.float32),
        pltpu.VMEM((64, DIM), jnp.float32),
        pltpu.VMEM((64, 16), jnp.float32),
        pltpu.VMEM_SHARED((NPAD, DIM), jnp.float32),
        pltpu.VMEM_SHARED((NPAD, 16), jnp.float32),
        pltpu.SemaphoreType.DMA,
    ],
    compiler_params=pltpu.CompilerParams(
        needs_layout_passes=False, use_tc_tiling_on_sc=False),
)
def _sc_scatter(y_hbm, src_hbm, dst_hbm, keep_hbm, acc_out, cnt_out,
                src_v, dst_v, csrc_v, cdst_v, keep_v, rows_v, ones_v, za_v,
                zc_v, acc_sh, cnt_sh, sem):
    cid = lax.axis_index("c")
    sid = lax.axis_index("s")
    wid = sid * NC + cid

    pltpu.sync_copy(src_hbm.at[wid], src_v)
    pltpu.sync_copy(dst_hbm.at[wid], dst_v)
    pltpu.sync_copy(keep_hbm, keep_v)

    zero16 = jnp.zeros((16,), jnp.float32)
    one16 = jnp.full((16,), 1.0, jnp.float32)

    def _fill(i, carry):
        for j in range(4):
            za_v[i, pl.ds(16 * j, 16)] = zero16
        zc_v[i] = zero16
        ones_v[i] = one16
        ones_v[64 + i] = one16
        return carry

    lax.fori_loop(0, 64, _fill, 0)

    base = sid * RPT
    for k in range(RPT // 64):
        pltpu.sync_copy(za_v, acc_sh.at[pl.ds(base + 64 * k, 64)])
        pltpu.sync_copy(zc_v, cnt_sh.at[pl.ds(base + 64 * k, 64)])

    lane = jnp.arange(16, dtype=jnp.int32)

    def _compact(j, off):
        for i in range(8):
            sl = pl.ds(16 * i, 16)
            s_idx = src_v[j, sl]
            d_idx = dst_v[j, sl]
            ks = plsc.load_gather(keep_v, [s_idx])
            kd = plsc.load_gather(keep_v, [d_idx])
            ok = (ks * kd) > 0
            cs = plsc.cumsum(ok.astype(jnp.int32))
            pos = off + cs - 1
            row = lax.shift_right_logical(pos, 7)
            col = lax.bitwise_and(pos, 127)
            plsc.store_scatter(csrc_v, [row, col], s_idx, mask=ok)
            plsc.store_scatter(cdst_v, [row, col], d_idx, mask=ok)
            off = off + jnp.max(cs)
        return off

    off = lax.fori_loop(0, NCHUNK, _compact, jnp.int32(0))

    for k in range(8):
        pos = off + 16 * k + lane
        row = lax.shift_right_logical(pos, 7)
        col = lax.bitwise_and(pos, 127)
        plsc.store_scatter(csrc_v, [row, col], jnp.zeros((16,), jnp.int32))
        plsc.store_scatter(cdst_v, [row, col], TRASH + lane)
    nchunks = lax.shift_right_logical(off + 127, 7)
    plsc.subcore_barrier()

    @pl.when(nchunks > 0)
    def _prime():
        pltpu.async_copy(y_hbm.at[csrc_v.at[0]], rows_v.at[0], sem)

    def _edges(j, carry):
        b = lax.rem(j, 2)
        pltpu.make_async_copy(y_hbm.at[csrc_v.at[j]], rows_v.at[b], sem).wait()

        @pl.when(j < nchunks - 1)
        def _prefetch():
            pltpu.async_copy(
                y_hbm.at[csrc_v.at[j + 1]], rows_v.at[lax.rem(j + 1, 2)], sem)

        pltpu.sync_copy(rows_v.at[b], acc_sh.at[cdst_v.at[j]], add=True)
        pltpu.sync_copy(ones_v, cnt_sh.at[cdst_v.at[j]], add=True)
        return carry

    lax.fori_loop(0, nchunks, _edges, 0)
    plsc.subcore_barrier()

    pltpu.sync_copy(acc_sh.at[pl.ds(base, RPT)],
                    acc_out.at[cid, pl.ds(base, RPT)])
    pltpu.sync_copy(cnt_sh.at[pl.ds(base, RPT)],
                    cnt_out.at[cid, pl.ds(base, RPT)])


@functools.partial(
    pl.kernel,
    out_type=jax.ShapeDtypeStruct((NPAD, DIM), jnp.float32),
    mesh=_mesh(),
    scratch_types=[
        pltpu.VMEM((RPW, DIM), jnp.float32),
        pltpu.VMEM((RPW, DIM), jnp.float32),
        pltpu.VMEM((RPW, 16), jnp.float32),
        pltpu.VMEM((RPW, 16), jnp.float32),
        pltpu.VMEM((RPW, DIM), jnp.float32),
        pltpu.VMEM((RPW, DIM), jnp.float32),
    ],
    compiler_params=pltpu.CompilerParams(
        needs_layout_passes=False, use_tc_tiling_on_sc=False),
)
def _sc_finalize(acc_hbm, cnt_hbm, xr_hbm, h_out, a0, a1, c0, c1, xr_v, h_v):
    cid = lax.axis_index("c")
    sid = lax.axis_index("s")
    wid = sid * NC + cid
    base = wid * RPW
    pltpu.sync_copy(acc_hbm.at[0, pl.ds(base, RPW)], a0)
    pltpu.sync_copy(acc_hbm.at[1, pl.ds(base, RPW)], a1)
    pltpu.sync_copy(cnt_hbm.at[0, pl.ds(base, RPW)], c0)
    pltpu.sync_copy(cnt_hbm.at[1, pl.ds(base, RPW)], c1)
    pltpu.sync_copy(xr_hbm.at[pl.ds(base, RPW)], xr_v)

    def _rows(i, carry):
        cnt = c0[i] + c1[i]
        inv = 1.0 / jnp.maximum(cnt, 1.0)
        for j in range(4):
            sl = pl.ds(16 * j, 16)
            v = (a0[i, sl] + a1[i, sl]) * inv + xr_v[i, sl]
            h_v[i, sl] = jnp.maximum(v, 0.0)
        return carry

    lax.fori_loop(0, RPW, _rows, 0)
    pltpu.sync_copy(h_v, h_out.at[pl.ds(base, RPW)])


@functools.partial(
    pl.kernel,
    out_type=(
        jax.ShapeDtypeStruct((NW, NUM_GRAPHS, DIM), jnp.float32),
        jax.ShapeDtypeStruct((NW, NUM_GRAPHS, DIM), jnp.float32),
    ),
    mesh=_mesh(),
    scratch_types=[
        pltpu.VMEM((RPW, DIM), jnp.float32),
        pltpu.VMEM((RPW,), jnp.float32),
        pltpu.VMEM((RPW,), jnp.int32),
        pltpu.VMEM((RPW,), jnp.int32),
        pltpu.VMEM((NUM_GRAPHS, DIM), jnp.float32),
        pltpu.VMEM((NUM_GRAPHS, DIM), jnp.float32),
    ],
    compiler_params=pltpu.CompilerParams(
        needs_layout_passes=False, use_tc_tiling_on_sc=False),
)
def _sc_readout(h_hbm, s_hbm, batch_hbm, keep_hbm, mx_out, sm_out,
                h_v, s_v, b_v, k_v, mx_v, sm_v):
    cid = lax.axis_index("c")
    sid = lax.axis_index("s")
    wid = sid * NC + cid
    base = wid * RPW
    pltpu.sync_copy(h_hbm.at[pl.ds(base, RPW)], h_v)
    pltpu.sync_copy(s_hbm.at[pl.ds(base, RPW)], s_v)
    pltpu.sync_copy(batch_hbm.at[pl.ds(base, RPW)], b_v)
    pltpu.sync_copy(keep_hbm.at[pl.ds(base, RPW)], k_v)

    neginf = jnp.full((16,), -jnp.inf, jnp.float32)
    zero16 = jnp.zeros((16,), jnp.float32)

    def _init(g, carry):
        for j in range(4):
            sl = pl.ds(16 * j, 16)
            mx_v[g, sl] = neginf
            sm_v[g, sl] = zero16
        return carry

    lax.fori_loop(0, NUM_GRAPHS, _init, 0)

    lane = jnp.arange(16, dtype=jnp.int32)

    def _rows(i, carry):
        isplat = jnp.zeros((16,), jnp.int32) + i
        bsplat = plsc.load_gather(b_v, [isplat])
        ksplat = plsc.load_gather(k_v, [isplat])
        ssplat = plsc.load_gather(s_v, [isplat])
        km = ksplat > 0
        for j in range(4):
            sl = pl.ds(16 * j, 16)
            col = 16 * j + lane
            val = h_v[i, sl] * ssplat
            cm = plsc.load_gather(mx_v, [bsplat, col])
            cs_ = plsc.load_gather(sm_v, [bsplat, col])
            vm = jnp.where(km, val, -jnp.inf)
            vs = jnp.where(km, val, 0.0)
            plsc.store_scatter(mx_v, [bsplat, col], jnp.maximum(cm, vm))
            plsc.store_scatter(sm_v, [bsplat, col], cs_ + vs)
        return carry

    lax.fori_loop(0, RPW, _rows, 0)
    pltpu.sync_copy(mx_v, mx_out.at[wid])
    pltpu.sync_copy(sm_v, sm_out.at[wid])


def _mm_body(x_ref, s_ref, wl_ref, bl_ref, wr_ref, y_ref, xr_ref):
    x = x_ref[...] * s_ref[...]
    y_ref[...] = jax.lax.dot_general(
        x, wl_ref[...], (((1,), (1,)), ((), ())),
        preferred_element_type=jnp.float32)
    xr_ref[...] = jax.lax.dot_general(
        x, wr_ref[...], (((1,), (1,)), ((), ())),
        preferred_element_type=jnp.float32) + bl_ref[...]


def _mm(x_pad, s_pad, Wl, bl, Wr):
    k = x_pad.shape[1]
    blk = NPAD // 4
    return pl.pallas_call(
        _mm_body,
        grid=(4,),
        in_specs=[
            pl.BlockSpec((blk, k), lambda i: (i, 0)),
            pl.BlockSpec((blk, 1), lambda i: (i, 0)),
            pl.BlockSpec((DIM, k), lambda i: (0, 0)),
            pl.BlockSpec((1, DIM), lambda i: (0, 0)),
            pl.BlockSpec((DIM, k), lambda i: (0, 0)),
        ],
        out_specs=[
            pl.BlockSpec((blk, DIM), lambda i: (i, 0)),
            pl.BlockSpec((blk, DIM), lambda i: (i, 0)),
        ],
        out_shape=[
            jax.ShapeDtypeStruct((NPAD, DIM), jnp.float32),
            jax.ShapeDtypeStruct((NPAD, DIM), jnp.float32),
        ],
    )(x_pad, s_pad[:, None], Wl, bl[None, :], Wr)


def _topk_keep(h, batch, p, node_mask, nstart):
    s = jnp.tanh((h @ p) / (jnp.linalg.norm(p) + 1e-16))
    bkey = jnp.where(node_mask, batch, NUM_GRAPHS)
    counts = jax.ops.segment_sum(
        node_mask.astype(jnp.int32), batch, num_segments=NUM_GRAPHS)
    starts = jnp.concatenate([jnp.zeros((1,), counts.dtype), jnp.cumsum(counts)])
    _, negs_sorted = jax.lax.sort((bkey, -s), num_keys=2)
    k_per = jnp.ceil(RATIO * counts).astype(jnp.int32)
    pos = starts[:NUM_GRAPHS] + jnp.maximum(k_per, 1) - 1
    thr = -negs_sorted[pos]
    thr_b = thr[batch]
    sgt = node_mask & (s > thr_b)
    tie = node_mask & (s == thr_b)
    c_gt = jax.ops.segment_sum(
        sgt.astype(jnp.int32), batch, num_segments=NUM_GRAPHS)
    tcum_ex = jnp.cumsum(tie.astype(jnp.int32)) - tie.astype(jnp.int32)
    base = tcum_ex[nstart[:NUM_GRAPHS]]
    rank_tie = tcum_ex - base[batch]
    keep = sgt | (tie & (rank_tie < (k_per - c_gt)[batch]))
    return s, keep, counts, k_per


def _mlp_kernel(r_ref, w1_ref, b1_ref, w2_ref, b2_ref, o_ref):
    h = jnp.maximum(r_ref[...] @ w1_ref[...].T + b1_ref[...], 0.0)
    o_ref[...] = h @ w2_ref[...].T + b2_ref[...]


def kernel(x, edge_index, batch, Wl1, bl1, Wr1, p1, Wl2, bl2, Wr2, p2,
           Wl3, bl3, Wr3, p3, Wl4, bl4, Wr4, p4, lw1, lb1, lw2, lb2):
    params = [(Wl1, bl1, Wr1, p1), (Wl2, bl2, Wr2, p2),
              (Wl3, bl3, Wr3, p3), (Wl4, bl4, Wr4, p4)]
    src = jnp.concatenate(
        [edge_index[0], jnp.zeros((EPAD - E,), jnp.int32)]).reshape(NW, NCHUNK, CHUNK)
    dst = jnp.concatenate(
        [edge_index[1], jnp.full((EPAD - E,), TRASH, jnp.int32)]).reshape(NW, NCHUNK, CHUNK)

    keep = jnp.ones((N,), jnp.bool_)
    nodes_per_graph = jax.ops.segment_sum(
        jnp.ones((N,), jnp.int32), batch, num_segments=NUM_GRAPHS)
    nstart = jnp.concatenate(
        [jnp.zeros((1,), jnp.int32), jnp.cumsum(nodes_per_graph)])
    batch_pad = jnp.concatenate([batch, jnp.zeros((NPAD - N,), jnp.int32)])
    cur_pad = jnp.concatenate([x, jnp.zeros((NPAD - N, x.shape[1]), jnp.float32)])
    s_pad = jnp.ones((NPAD,), jnp.float32)
    r = None
    for (Wl, bl, Wr, p) in params:
        y_pad, xr_pad = _mm(cur_pad, s_pad, Wl, bl, Wr)
        keep_pad = jnp.concatenate(
            [keep.astype(jnp.int32), jnp.zeros((NPAD - N,), jnp.int32)])
        acc, cnt = _sc_scatter(y_pad, src, dst, keep_pad)
        h_pad = _sc_finalize(acc, cnt, xr_pad)
        h = h_pad[:N]
        s, keep, counts, k_per = _topk_keep(h, batch, p, keep, nstart)
        cur_pad = jnp.concatenate(
            [h, jnp.zeros((NPAD - N, DIM), jnp.float32)])
        s_pad = jnp.concatenate([s, jnp.zeros((NPAD - N,), jnp.float32)])
        knew_pad = jnp.concatenate(
            [keep.astype(jnp.int32), jnp.zeros((NPAD - N,), jnp.int32)])
        mxp, smp = _sc_readout(cur_pad, s_pad, batch_pad, knew_pad)
        gm = jnp.max(mxp, axis=0)
        gs = jnp.sum(smp, axis=0)
        cnt = jnp.where(counts > 0, k_per, 0).astype(jnp.float32)
        ga = gs / jnp.maximum(cnt, 1.0)[:, None]
        ro = jnp.concatenate([gm, ga], axis=1)
        r = ro if r is None else r + ro

    out = pl.pallas_call(
        _mlp_kernel,
        out_shape=jax.ShapeDtypeStruct((NUM_GRAPHS, lw2.shape[0]), jnp.float32),
    )(r, lw1, lb1, lw2, lb2)
    return out
